# Initial kernel scaffold; baseline (speedup 1.0000x reference)
import functools

import jax
import jax.numpy as jnp
from jax import lax
from jax.experimental import pallas as pl
from jax.experimental.pallas import tpu as pltpu
from jax.experimental.pallas import tpu_sc as plsc

_N = 10000
_C = 128
_NPAD = 10240
_EROWS = (320000 + _NPAD) // 128
_CH = 4
_SLICE = _NPAD // 16


def _mesh():
    return plsc.VectorSubcoreMesh(core_axis_name="c", subcore_axis_name="s")


def _embed_body(x_ref, w_ref, av_ref, bv_ref, h_ref, as_ref, ad_ref):
    h = jnp.dot(x_ref[...], w_ref[...], preferred_element_type=jnp.float32)
    h_ref[...] = h
    as_ref[...] = jnp.sum(h * av_ref[...][None, :], axis=1, keepdims=True)
    ad_ref[...] = jnp.sum(h * bv_ref[...][None, :], axis=1, keepdims=True)


def _tc_embed(x, w, av, bv):
    return pl.pallas_call(
        _embed_body,
        out_shape=(
            jax.ShapeDtypeStruct((_NPAD, _C), jnp.float32),
            jax.ShapeDtypeStruct((_NPAD, 1), jnp.float32),
            jax.ShapeDtypeStruct((_NPAD, 1), jnp.float32),
        ),
    )(x, w, av, bv)


def _mid_body(p0_ref, p1_ref, b_ref, w_ref, av_ref, bv_ref,
              h_ref, as_ref, ad_ref):
    x2 = jnp.maximum(p0_ref[...] + p1_ref[...] + b_ref[...][None, :], 0.0)
    h = jnp.dot(x2, w_ref[...], preferred_element_type=jnp.float32)
    h_ref[...] = h
    as_ref[...] = jnp.sum(h * av_ref[...][None, :], axis=1, keepdims=True)
    ad_ref[...] = jnp.sum(h * bv_ref[...][None, :], axis=1, keepdims=True)


def _tc_mid(p0, p1, b, w, av, bv):
    return pl.pallas_call(
        _mid_body,
        out_shape=(
            jax.ShapeDtypeStruct((_NPAD, _C), jnp.float32),
            jax.ShapeDtypeStruct((_NPAD, 1), jnp.float32),
            jax.ShapeDtypeStruct((_NPAD, 1), jnp.float32),
        ),
    )(p0, p1, b, w, av, bv)


def _final_body(p0_ref, p1_ref, b_ref, o_ref):
    o_ref[...] = p0_ref[...] + p1_ref[...] + b_ref[...][None, :]


def _tc_final(p0, p1, b):
    return pl.pallas_call(
        _final_body,
        out_shape=jax.ShapeDtypeStruct((_NPAD, _C), jnp.float32),
    )(p0, p1, b)


def _sc_pass1(alpha_src, alpha_dst, src2, dst2):
    @functools.partial(
        pl.kernel,
        mesh=_mesh(),
        out_type=(
            jax.ShapeDtypeStruct((_EROWS, 128), jnp.float32),
            jax.ShapeDtypeStruct((2, _NPAD), jnp.float32),
        ),
        scratch_types=[
            pltpu.VMEM((_NPAD,), jnp.float32),
            pltpu.VMEM((_NPAD,), jnp.float32),
            pltpu.VMEM((_CH, 128), jnp.int32),
            pltpu.VMEM((_CH, 128), jnp.int32),
            pltpu.VMEM((_CH, 128), jnp.float32),
            pltpu.VMEM((_SLICE,), jnp.float32),
            pltpu.VMEM_SHARED((_NPAD,), jnp.float32),
        ],
    )
    def k(as_hbm, ad_hbm, src_hbm, dst_hbm, ex_hbm, sp_hbm,
          as_v, ad_v, src_v, dst_v, ex_v, zero_v, s_sh):
        cid = lax.axis_index("c")
        sid = lax.axis_index("s")
        wid = sid * 2 + cid

        pltpu.sync_copy(as_hbm, as_v)
        pltpu.sync_copy(ad_hbm, ad_v)

        def _mx(i, acc):
            return jnp.maximum(acc, as_v[pl.ds(i * 16, 16)])

        mx = lax.fori_loop(0, _NPAD // 16, _mx,
                           jnp.full((16,), -3.0e38, jnp.float32))
        asmax = jnp.max(mx)

        @pl.loop(0, _SLICE // 16)
        def _(i):
            zero_v[pl.ds(i * 16, 16)] = jnp.zeros((16,), jnp.float32)

        pltpu.sync_copy(zero_v, s_sh.at[pl.ds(sid * _SLICE, _SLICE)])
        plsc.subcore_barrier()

        @pl.loop(wid * _CH, _EROWS, step=32 * _CH)
        def _(r0):
            pltpu.sync_copy(src_hbm.at[pl.ds(r0, _CH)], src_v)
            pltpu.sync_copy(dst_hbm.at[pl.ds(r0, _CH)], dst_v)
            for j in range(_CH):
                srow = src_v.at[j]
                drow = dst_v.at[j]
                erow = ex_v.at[j]
                for c in range(8):
                    sl = pl.ds(c * 16, 16)
                    av = plsc.load_gather(as_v, [srow[sl]])
                    bv = plsc.load_gather(ad_v, [drow[sl]])
                    z = av + bv
                    e = jnp.where(z > 0, z, 0.2 * z)
                    zu = asmax + bv
                    mub = jnp.where(zu > 0, zu, 0.2 * zu)
                    erow[sl] = jnp.exp(e - mub)
            pltpu.sync_copy(ex_v, ex_hbm.at[pl.ds(r0, _CH)])
            for j in range(_CH):
                pltpu.sync_copy(ex_v.at[j], s_sh.at[dst_v.at[j]], add=True)

        plsc.subcore_barrier()
        pltpu.sync_copy(s_sh.at[pl.ds(sid * _SLICE, _SLICE)],
                        sp_hbm.at[cid, pl.ds(sid * _SLICE, _SLICE)])

    return k(alpha_src, alpha_dst, src2, dst2)


def _sc_pass2(h, ex2, sp, src2, dst2):
    @functools.partial(
        pl.kernel,
        mesh=_mesh(),
        out_type=jax.ShapeDtypeStruct((2, _NPAD, _C), jnp.float32),
        scratch_types=[
            pltpu.VMEM((_NPAD,), jnp.float32),
            pltpu.VMEM((_NPAD,), jnp.float32),
            pltpu.VMEM((_CH, 128), jnp.int32),
            pltpu.VMEM((_CH, 128), jnp.int32),
            pltpu.VMEM((_CH, 128), jnp.float32),
            pltpu.VMEM((_CH, 128), jnp.float32),
            pltpu.VMEM((2, 128, _C), jnp.float32),
            pltpu.VMEM_SHARED((_NPAD, _C), jnp.float32),
            pltpu.SemaphoreType.DMA,
            pltpu.SemaphoreType.DMA,
        ],
    )
    def k(h_hbm, ex_hbm, sp_hbm, src_hbm, dst_hbm, out_hbm,
          r_v, t_v, src_v, dst_v, ex_v, att_v, rows_v, acc_sh, sem0, sem1):
        cid = lax.axis_index("c")
        sid = lax.axis_index("s")
        wid = sid * 2 + cid

        pltpu.sync_copy(sp_hbm.at[0], r_v)
        pltpu.sync_copy(sp_hbm.at[1], t_v)

        @pl.loop(0, _NPAD // 16)
        def _(i):
            sl = pl.ds(i * 16, 16)
            r_v[sl] = 1.0 / (r_v[sl] + t_v[sl] + 1e-16)

        zrows = rows_v.at[0]

        @pl.loop(0, 128)
        def _(rr):
            zrow = zrows.at[rr]
            for c in range(8):
                zrow[pl.ds(c * 16, 16)] = jnp.zeros((16,), jnp.float32)

        for t in range(5):
            pltpu.sync_copy(zrows,
                            acc_sh.at[pl.ds(sid * _SLICE + t * 128, 128)])
        plsc.subcore_barrier()

        sems = (sem0, sem1)

        @pl.loop(wid * _CH, _EROWS, step=32 * _CH)
        def _(r0):
            pltpu.sync_copy(src_hbm.at[pl.ds(r0, _CH)], src_v)
            pltpu.sync_copy(dst_hbm.at[pl.ds(r0, _CH)], dst_v)
            pltpu.sync_copy(ex_hbm.at[pl.ds(r0, _CH)], ex_v)
            cps = [
                pltpu.make_async_copy(h_hbm.at[src_v.at[j]],
                                      rows_v.at[j % 2], sems[j % 2])
                for j in range(_CH)
            ]
            cps[0].start()
            for j in range(_CH):
                drow = dst_v.at[j]
                erow = ex_v.at[j]
                arow = att_v.at[j]
                for c in range(8):
                    sl = pl.ds(c * 16, 16)
                    rv = plsc.load_gather(r_v, [drow[sl]])
                    arow[sl] = erow[sl] * rv
                if j + 1 < _CH:
                    cps[j + 1].start()
                cps[j].wait()
                rb = rows_v.at[j % 2]

                @pl.loop(0, 128)
                def _(rr):
                    a = arow[rr]
                    rrow = rb.at[rr]
                    for c in range(8):
                        sl = pl.ds(c * 16, 16)
                        rrow[sl] = rrow[sl] * a

                pltpu.sync_copy(rb, acc_sh.at[drow], add=True)

        plsc.subcore_barrier()
        pltpu.sync_copy(acc_sh.at[pl.ds(sid * _SLICE, _SLICE)],
                        out_hbm.at[cid, pl.ds(sid * _SLICE, _SLICE)])

    return k(h, ex2, sp, src2, dst2)


def _layer(h, a_s, a_d, src2, dst2):
    ex, sp = _sc_pass1(a_s.reshape(-1), a_d.reshape(-1), src2, dst2)
    return _sc_pass2(h, ex, sp, src2, dst2)


@jax.jit
def kernel(x, edge_index, W1, a_src1, a_dst1, b1, W2, a_src2, a_dst2, b2):
    ei = edge_index.astype(jnp.int32)
    loops = jnp.arange(_NPAD, dtype=jnp.int32)
    src2 = jnp.concatenate([ei[0], loops]).reshape(_EROWS, 128)
    dst2 = jnp.concatenate([ei[1], loops]).reshape(_EROWS, 128)
    x_pad = jnp.pad(x, ((0, _NPAD - _N), (0, 0)))

    h1, as1, ad1 = _tc_embed(x_pad, W1, a_src1[0], a_dst1[0])
    p1 = _layer(h1, as1, ad1, src2, dst2)
    h2, as2, ad2 = _tc_mid(p1[0], p1[1], b1, W2, a_src2[0], a_dst2[0])
    p2 = _layer(h2, as2, ad2, src2, dst2)
    out = _tc_final(p2[0], p2[1], b2)
    return out[:_N]


# speedup vs baseline: 31.3405x; 31.3405x over previous
import dataclasses
import functools

import jax
import jax.numpy as jnp
from jax import lax
from jax.experimental import pallas as pl
from jax.experimental.pallas import tpu as pltpu
from jax.experimental.pallas import tpu_sc as plsc

_N = 10000
_C = 128
_NPAD = 10240
_EROWS = (320000 + _NPAD) // 128
_CH = 4
_SLICE = _NPAD // 16


def _mesh():
    return plsc.VectorSubcoreMesh(core_axis_name="c", subcore_axis_name="s")


def _sc_params():
    cp = pltpu.CompilerParams()
    if "needs_layout_passes" in pltpu.CompilerParams.__dataclass_fields__:
        cp = dataclasses.replace(cp, needs_layout_passes=False)
    return cp


def _embed_body(x_ref, w_ref, av_ref, bv_ref, h_ref, as_ref, ad_ref):
    h = jnp.dot(x_ref[...], w_ref[...], preferred_element_type=jnp.float32)
    h_ref[...] = h
    as_ref[...] = jnp.sum(h * av_ref[...][None, :], axis=1, keepdims=True)
    ad_ref[...] = jnp.sum(h * bv_ref[...][None, :], axis=1, keepdims=True)


def _tc_embed(x, w, av, bv):
    return pl.pallas_call(
        _embed_body,
        out_shape=(
            jax.ShapeDtypeStruct((_NPAD, _C), jnp.float32),
            jax.ShapeDtypeStruct((_NPAD, 1), jnp.float32),
            jax.ShapeDtypeStruct((_NPAD, 1), jnp.float32),
        ),
    )(x, w, av, bv)


def _mid_body(p0_ref, p1_ref, b_ref, w_ref, av_ref, bv_ref,
              h_ref, as_ref, ad_ref):
    x2 = jnp.maximum(p0_ref[...] + p1_ref[...] + b_ref[...][None, :], 0.0)
    h = jnp.dot(x2, w_ref[...], preferred_element_type=jnp.float32)
    h_ref[...] = h
    as_ref[...] = jnp.sum(h * av_ref[...][None, :], axis=1, keepdims=True)
    ad_ref[...] = jnp.sum(h * bv_ref[...][None, :], axis=1, keepdims=True)


def _tc_mid(p0, p1, b, w, av, bv):
    return pl.pallas_call(
        _mid_body,
        out_shape=(
            jax.ShapeDtypeStruct((_NPAD, _C), jnp.float32),
            jax.ShapeDtypeStruct((_NPAD, 1), jnp.float32),
            jax.ShapeDtypeStruct((_NPAD, 1), jnp.float32),
        ),
    )(p0, p1, b, w, av, bv)


def _final_body(p0_ref, p1_ref, b_ref, o_ref):
    o_ref[...] = p0_ref[...] + p1_ref[...] + b_ref[...][None, :]


def _tc_final(p0, p1, b):
    return pl.pallas_call(
        _final_body,
        out_shape=jax.ShapeDtypeStruct((_NPAD, _C), jnp.float32),
    )(p0, p1, b)


def _sc_pass1(alpha_src, alpha_dst, src2, dst2):
    @functools.partial(
        pl.kernel,
        mesh=_mesh(),
        compiler_params=_sc_params(),
        out_type=(
            jax.ShapeDtypeStruct((_EROWS, 128), jnp.float32),
            jax.ShapeDtypeStruct((2, _NPAD), jnp.float32),
        ),
        scratch_types=[
            pltpu.VMEM((_NPAD,), jnp.float32),
            pltpu.VMEM((_NPAD,), jnp.float32),
            pltpu.VMEM((_CH, 128), jnp.int32),
            pltpu.VMEM((_CH, 128), jnp.int32),
            pltpu.VMEM((_CH, 128), jnp.float32),
            pltpu.VMEM((_SLICE,), jnp.float32),
            pltpu.VMEM_SHARED((_NPAD,), jnp.float32),
        ],
    )
    def k(as_hbm, ad_hbm, src_hbm, dst_hbm, ex_hbm, sp_hbm,
          as_v, ad_v, src_v, dst_v, ex_v, zero_v, s_sh):
        cid = lax.axis_index("c")
        sid = lax.axis_index("s")
        wid = sid * 2 + cid

        pltpu.sync_copy(as_hbm, as_v)
        pltpu.sync_copy(ad_hbm, ad_v)

        def _mx(i, acc):
            return jnp.maximum(acc, as_v[pl.ds(i * 16, 16)])

        mx = lax.fori_loop(0, _NPAD // 16, _mx,
                           jnp.full((16,), -3.0e38, jnp.float32))
        asmax = jnp.max(mx)

        @pl.loop(0, _SLICE // 16)
        def _(i):
            zero_v[pl.ds(i * 16, 16)] = jnp.zeros((16,), jnp.float32)

        pltpu.sync_copy(zero_v, s_sh.at[pl.ds(sid * _SLICE, _SLICE)])
        plsc.subcore_barrier()

        @pl.loop(wid * _CH, _EROWS, step=32 * _CH)
        def _(r0):
            pltpu.sync_copy(src_hbm.at[pl.ds(r0, _CH)], src_v)
            pltpu.sync_copy(dst_hbm.at[pl.ds(r0, _CH)], dst_v)
            for j in range(_CH):
                srow = src_v.at[j]
                drow = dst_v.at[j]
                erow = ex_v.at[j]
                for c in range(8):
                    sl = pl.ds(c * 16, 16)
                    av = plsc.load_gather(as_v, [srow[sl]])
                    bv = plsc.load_gather(ad_v, [drow[sl]])
                    z = av + bv
                    e = jnp.where(z > 0, z, 0.2 * z)
                    zu = asmax + bv
                    mub = jnp.where(zu > 0, zu, 0.2 * zu)
                    erow[sl] = jnp.exp(e - mub)
            pltpu.sync_copy(ex_v, ex_hbm.at[pl.ds(r0, _CH)])
            for j in range(_CH):
                pltpu.sync_copy(ex_v.at[j], s_sh.at[dst_v.at[j]], add=True)

        plsc.subcore_barrier()
        pltpu.sync_copy(s_sh.at[pl.ds(sid * _SLICE, _SLICE)],
                        sp_hbm.at[cid, pl.ds(sid * _SLICE, _SLICE)])

    return k(alpha_src, alpha_dst, src2, dst2)


def _sc_pass2(h, ex2, sp, src2, dst2):
    @functools.partial(
        pl.kernel,
        mesh=_mesh(),
        compiler_params=_sc_params(),
        out_type=jax.ShapeDtypeStruct((2, _NPAD, _C), jnp.float32),
        scratch_types=[
            pltpu.VMEM((_NPAD,), jnp.float32),
            pltpu.VMEM((_SLICE,), jnp.float32),
            pltpu.VMEM((_CH, 128), jnp.int32),
            pltpu.VMEM((_CH, 128), jnp.int32),
            pltpu.VMEM((_CH, 128), jnp.float32),
            pltpu.VMEM((_CH, 128), jnp.float32),
            pltpu.VMEM((2, 128, _C), jnp.float32),
            pltpu.VMEM_SHARED((_NPAD, _C), jnp.float32),
            pltpu.SemaphoreType.DMA,
            pltpu.SemaphoreType.DMA,
        ],
    )
    def k(h_hbm, ex_hbm, sp_hbm, src_hbm, dst_hbm, out_hbm,
          r_v, t_v, src_v, dst_v, ex_v, att_v, rows_v, acc_sh, sem0, sem1):
        cid = lax.axis_index("c")
        sid = lax.axis_index("s")
        wid = sid * 2 + cid

        pltpu.sync_copy(sp_hbm.at[0], r_v)

        @pl.loop(0, _NPAD // _SLICE)
        def _(w):
            pltpu.sync_copy(sp_hbm.at[1, pl.ds(w * _SLICE, _SLICE)], t_v)

            @pl.loop(0, _SLICE // 16)
            def _(i):
                sl = pl.ds(w * _SLICE + i * 16, 16)
                r_v[sl] = 1.0 / (r_v[sl] + t_v[pl.ds(i * 16, 16)] + 1e-16)

        zrows = rows_v.at[0]

        @pl.loop(0, 128)
        def _(rr):
            zrow = zrows.at[rr]
            for c in range(8):
                zrow[pl.ds(c * 16, 16)] = jnp.zeros((16,), jnp.float32)

        for t in range(5):
            pltpu.sync_copy(zrows,
                            acc_sh.at[pl.ds(sid * _SLICE + t * 128, 128)])
        plsc.subcore_barrier()

        sems = (sem0, sem1)

        @pl.loop(wid * _CH, _EROWS, step=32 * _CH)
        def _(r0):
            pltpu.sync_copy(src_hbm.at[pl.ds(r0, _CH)], src_v)
            pltpu.sync_copy(dst_hbm.at[pl.ds(r0, _CH)], dst_v)
            pltpu.sync_copy(ex_hbm.at[pl.ds(r0, _CH)], ex_v)
            cps = [
                pltpu.make_async_copy(h_hbm.at[src_v.at[j]],
                                      rows_v.at[j % 2], sems[j % 2])
                for j in range(_CH)
            ]
            cps[0].start()
            for j in range(_CH):
                drow = dst_v.at[j]
                erow = ex_v.at[j]
                arow = att_v.at[j]
                for c in range(8):
                    sl = pl.ds(c * 16, 16)
                    rv = plsc.load_gather(r_v, [drow[sl]])
                    arow[sl] = erow[sl] * rv
                if j + 1 < _CH:
                    cps[j + 1].start()
                cps[j].wait()
                rb = rows_v.at[j % 2]

                @pl.loop(0, 8)
                def _(g):
                    a16 = arow[pl.ds(g * 16, 16)]
                    for i in range(16):
                        ai = a16[i]
                        rrow = rb.at[g * 16 + i]
                        for c in range(8):
                            sl = pl.ds(c * 16, 16)
                            rrow[sl] = rrow[sl] * ai

                pltpu.sync_copy(rb, acc_sh.at[drow], add=True)

        plsc.subcore_barrier()
        pltpu.sync_copy(acc_sh.at[pl.ds(sid * _SLICE, _SLICE)],
                        out_hbm.at[cid, pl.ds(sid * _SLICE, _SLICE)])

    return k(h, ex2, sp, src2, dst2)


def _layer(h, a_s, a_d, src2, dst2):
    ex, sp = _sc_pass1(a_s.reshape(-1), a_d.reshape(-1), src2, dst2)
    return _sc_pass2(h, ex, sp, src2, dst2)


@jax.jit
def kernel(x, edge_index, W1, a_src1, a_dst1, b1, W2, a_src2, a_dst2, b2):
    ei = edge_index.astype(jnp.int32)
    loops = jnp.arange(_NPAD, dtype=jnp.int32)
    src2 = jnp.concatenate([ei[0], loops]).reshape(_EROWS, 128)
    dst2 = jnp.concatenate([ei[1], loops]).reshape(_EROWS, 128)
    x_pad = jnp.pad(x, ((0, _NPAD - _N), (0, 0)))

    h1, as1, ad1 = _tc_embed(x_pad, W1, a_src1[0], a_dst1[0])
    p1 = _layer(h1, as1, ad1, src2, dst2)
    h2, as2, ad2 = _tc_mid(p1[0], p1[1], b1, W2, a_src2[0], a_dst2[0])
    p2 = _layer(h2, as2, ad2, src2, dst2)
    out = _tc_final(p2[0], p2[1], b2)
    return out[:_N]
